# baseline (speedup 1.0000x reference)
import functools

import jax
import jax.numpy as jnp
from jax.experimental import pallas as pl
from jax.experimental.pallas import tpu as pltpu

_C = 80
_EPS_LS = 0.05
_KMAX = 10
_NCHUNK = 4


def _bce_logits(x, t):
    return jnp.maximum(x, 0.0) - x * t + jnp.log1p(jnp.exp(-jnp.abs(x)))


def _atan(z):
    s = jnp.abs(z)
    inv = s > 1.0
    t = jnp.where(inv, 1.0 / jnp.maximum(s, 1e-30), s)
    red = t > 0.41421356
    u = jnp.where(red, (t - 1.0) / (t + 1.0), t)
    u2 = u * u
    p = u * (1.0 + u2 * (-1.0 / 3 + u2 * (1.0 / 5 + u2 * (-1.0 / 7 + u2 * (
        1.0 / 9 + u2 * (-1.0 / 11 + u2 * (1.0 / 13)))))))
    r = jnp.where(red, jnp.pi / 4 + p, p)
    r = jnp.where(inv, jnp.pi / 2 - r, r)
    return jnp.where(z < 0.0, -r, r)


def _lane_slot(vals, slot0, T):
    lane = jax.lax.broadcasted_iota(jnp.int32, (T, 128), 1)
    out = jnp.zeros((T, 128), jnp.float32)
    for q, v in enumerate(vals):
        out = out + jnp.where(lane == slot0 + q, v, 0.0)
    return out


def _phase1_kernel(pbo_ref, pcls_ref, anch_ref, tgt_ref, toh_ref,
                   cr_ref, im_ref, cand_ref):
    f32 = jnp.float32
    T = tgt_ref.shape[0]
    Ac = anch_ref.shape[1]

    img = pl.program_id(0).astype(f32)

    pbo = pbo_ref[0]
    px1 = pbo[0:1, :]
    py1 = pbo[1:2, :]
    px2 = pbo[2:3, :]
    py2 = pbo[3:4, :]
    pobj = pbo[4:5, :]

    gx = anch_ref[0:1, :]
    gy = anch_ref[1:2, :]
    stride = anch_ref[2:3, :]
    xc = (gx + 0.5) * stride
    yc = (gy + 0.5) * stride

    tgt = tgt_ref[...]
    tbi = tgt[:, 0:1]
    tx1 = tgt[:, 2:3]
    ty1 = tgt[:, 3:4]
    tx2 = tgt[:, 4:5]
    ty2 = tgt[:, 5:6]
    tmask = tbi == img

    in_box = (jnp.minimum(jnp.minimum(xc - tx1, yc - ty1),
                          jnp.minimum(tx2 - xc, ty2 - yc))
              > 0.0) & tmask
    cxg = (tx1 + tx2) * 0.5
    cyg = (ty1 + ty2) * 0.5
    in_ctr = (jnp.maximum(jnp.abs(xc - cxg), jnp.abs(yc - cyg))
              < 2.5 * stride) & tmask
    fg = (jnp.sum(in_box.astype(f32), axis=0, keepdims=True) > 0.0) | (
        jnp.sum(in_ctr.astype(f32), axis=0, keepdims=True) > 0.0)
    is_matched = in_box & in_ctr

    area_t = (tx2 - tx1) * (ty2 - ty1)
    area_p = (px2 - px1) * (py2 - py1)
    iw = jnp.clip(jnp.minimum(tx2, px2) - jnp.maximum(tx1, px1), 0.0)
    ih = jnp.clip(jnp.minimum(ty2, py2) - jnp.maximum(ty1, py1), 0.0)
    inter = iw * ih
    iou = inter / (area_t + area_p - inter + 1e-12)

    cls = pcls_ref[0]
    clsp = jax.nn.sigmoid(pobj) * jax.nn.sigmoid(cls)
    cs = jnp.sqrt(jnp.clip(clsp, 1e-12, 1.0))
    lcs = jnp.log(cs + 1e-8)
    l1cs = jnp.log(1.0 - cs + 1e-8)
    s1 = jnp.sum(l1cs, axis=0, keepdims=True)
    d = lcs - l1cs
    sel = jnp.dot(toh_ref[...], d, preferred_element_type=f32)
    bce = -(sel + s1)

    cost = bce - 3.0 * jnp.log(iou + 1e-8) \
        + 100000.0 * (1.0 - is_matched.astype(f32))

    iou_m = jnp.where(fg, iou, 0.0)
    cost_r = jnp.where(fg, cost, 1e9)
    cr_ref[0] = cost_r
    im_ref[0] = (iou > 0.0).astype(jnp.int8)

    idx = jax.lax.broadcasted_iota(jnp.int32, (T, Ac), 1).astype(f32)

    x = iou_m
    ivals, icnts = [], []
    for _ in range(_KMAX):
        m = jnp.max(x, axis=1, keepdims=True)
        eq = x == m
        cnt = jnp.sum(eq.astype(f32), axis=1, keepdims=True)
        ivals.append(m)
        icnts.append(cnt)
        x = jnp.where(eq, -1e30, x)

    y = cost_r
    cvals, cidx = [], []
    for _ in range(_KMAX):
        m = jnp.min(y, axis=1, keepdims=True)
        eq = y == m
        imn = jnp.min(jnp.where(eq, idx, float(Ac)), axis=1, keepdims=True)
        cvals.append(m)
        cidx.append(imn)
        y = jnp.where(idx == imn, 1e30, y)

    amc = jnp.sum(fg.astype(f32))
    cand = (_lane_slot(ivals, 0, T)
            + _lane_slot(cvals, 16, T)
            + _lane_slot(cidx, 32, T)
            + _lane_slot([jnp.full((T, 1), amc, f32)], 48, T)
            + _lane_slot(icnts, 64, T))
    cand_ref[0, 0] = cand


def _phase2_kernel(cr_ref, im_ref, cand_ref, pbo_ref, pcls_ref, anch_ref,
                   tgt_ref, out_ref):
    f32 = jnp.float32
    T = tgt_ref.shape[0]
    Ac = anch_ref.shape[1]
    K = cand_ref.shape[1]

    img = pl.program_id(0).astype(f32)
    chunk = pl.program_id(1)

    tgt = tgt_ref[...]
    tmask = tgt[:, 0:1] == img

    cost_r = cr_ref[0]
    ioupos = im_ref[0] != 0

    iou_c = jnp.concatenate([cand_ref[0, k][:, 0:_KMAX] for k in range(K)],
                            axis=1)
    icnt_c = jnp.concatenate([cand_ref[0, k][:, 64:64 + _KMAX]
                              for k in range(K)], axis=1)
    cv_c = jnp.concatenate([cand_ref[0, k][:, 16:16 + _KMAX]
                            for k in range(K)], axis=1)
    base = jnp.concatenate(
        [jnp.full((T, _KMAX), float(k * Ac), f32) for k in range(K)], axis=1)
    ci_c = jnp.concatenate([cand_ref[0, k][:, 32:32 + _KMAX]
                            for k in range(K)], axis=1) + base
    am = functools.reduce(
        lambda a, b: a + b,
        [cand_ref[0, k][0:1, 48:49] for k in range(K)])

    x = iou_c
    acc = jnp.zeros((T, 1), f32)
    krem = jnp.full((T, 1), float(_KMAX), f32)
    for _ in range(_KMAX):
        m = jnp.max(x, axis=1, keepdims=True)
        eq = x == m
        csum = jnp.sum(jnp.where(eq, icnt_c, 0.0), axis=1, keepdims=True)
        take = jnp.minimum(csum, krem)
        acc = acc + m * take
        krem = krem - take
        x = jnp.where(eq, -1e30, x)
    dks = jnp.clip(jnp.floor(acc), 1.0, am)

    y = cv_c
    tv = jnp.full((T, 1), -1e30, f32)
    ti = jnp.full((T, 1), -1.0, f32)
    for k in range(_KMAX):
        m = jnp.min(y, axis=1, keepdims=True)
        eq = y == m
        iv = jnp.min(jnp.where(eq, ci_c, 1e9), axis=1, keepdims=True)
        hit = dks == float(k + 1)
        tv = jnp.where(hit, m, tv)
        ti = jnp.where(hit, iv, ti)
        y = jnp.where(ci_c == iv, 1e30, y)

    idxg = (jax.lax.broadcasted_iota(jnp.int32, (T, Ac), 1)
            + chunk * Ac).astype(f32)
    matching = ((cost_r < tv) | ((cost_r == tv) & (idxg <= ti))) & tmask

    mf = matching.astype(f32)
    agt = jnp.sum(mf, axis=0, keepdims=True)
    conflict = agt > 1.0
    cost_b = jnp.where(tmask, cost_r, 1e12)
    mb = jnp.min(cost_b, axis=0, keepdims=True)
    tio = jax.lax.broadcasted_iota(jnp.int32, (T, Ac), 0).astype(f32)
    best = jnp.min(jnp.where(cost_b == mb, tio, float(T)),
                   axis=0, keepdims=True)
    onehot_best = (tio == best).astype(f32)
    mf = jnp.where(conflict, onehot_best, mf)

    fgf = (jnp.sum(mf, axis=0, keepdims=True) > 0.0).astype(f32)
    piou_pos = jnp.sum(jnp.where(ioupos, mf, 0.0), axis=0,
                       keepdims=True) > 0.0

    tattr = jax.lax.dot_general(tgt, mf, (((0,), (0,)), ((), ())),
                                preferred_element_type=f32)
    cidm = tattr[1:2, :]
    rx1 = tattr[2:3, :]
    ry1 = tattr[3:4, :]
    rx2 = tattr[4:5, :]
    ry2 = tattr[5:6, :]

    pbo = pbo_ref[0]
    px1 = pbo[0:1, :]
    py1 = pbo[1:2, :]
    px2 = pbo[2:3, :]
    py2 = pbo[3:4, :]
    pobj = pbo[4:5, :]
    valid = anch_ref[3:4, :]

    eps = 1e-7
    pa = (px2 - px1) * (py2 - py1)
    ta = (rx2 - rx1) * (ry2 - ry1)
    ciw = jnp.clip(jnp.minimum(px2, rx2) - jnp.maximum(px1, rx1), 0.0)
    cih = jnp.clip(jnp.minimum(py2, ry2) - jnp.maximum(py1, ry1), 0.0)
    cinter = ciw * cih
    cunion = pa + ta - cinter + eps
    ciou_i = cinter / cunion
    cw = jnp.maximum(px2, rx2) - jnp.minimum(px1, rx1)
    ch = jnp.maximum(py2, ry2) - jnp.minimum(py1, ry1)
    c2 = cw * cw + ch * ch + eps
    rho2 = ((px1 + px2 - rx1 - rx2) ** 2 + (py1 + py2 - ry1 - ry2) ** 2) / 4.0
    v = (4.0 / (jnp.pi ** 2)) * (
        _atan((rx2 - rx1) / (ry2 - ry1 + eps))
        - _atan((px2 - px1) / (py2 - py1 + eps))) ** 2
    alpha = v / (1.0 - ciou_i + v + eps)
    ciou_full = ciou_i - rho2 / c2 - alpha * v
    lbox_i = jnp.sum((1.0 - ciou_full) * fgf)

    cls = pcls_ref[0]
    crow = jax.lax.broadcasted_iota(jnp.int32, (_C, Ac), 0).astype(f32)
    cls_t = jnp.where((crow == cidm) & piou_pos,
                      1.0 - _EPS_LS, _EPS_LS / (_C - 1))
    lcls_i = jnp.sum(jnp.sum(_bce_logits(cls, cls_t), axis=0,
                             keepdims=True) * fgf)
    lobj_i = jnp.sum(_bce_logits(pobj, fgf) * valid)
    nfg_i = jnp.sum(fgf)

    lane = jax.lax.broadcasted_iota(jnp.int32, (1, 128), 1)
    out_ref[0, 0] = (jnp.where(lane == 0, nfg_i, 0.0)
                     + jnp.where(lane == 1, lbox_i, 0.0)
                     + jnp.where(lane == 2, lcls_i, 0.0)
                     + jnp.where(lane == 3, lobj_i, 0.0))


@jax.jit
def kernel(preds, grid_mask, stride_mask, targets):
    f32 = jnp.float32
    N, A, _ = preds.shape
    T = targets.shape[0]
    K = _NCHUNK
    Ap = ((A + 128 * K - 1) // (128 * K)) * (128 * K)
    Ac = Ap // K

    pt = jnp.transpose(preds, (0, 2, 1))
    pbo = jnp.pad(pt[:, :5, :], ((0, 0), (0, 3), (0, Ap - A)))
    pcls = jnp.pad(pt[:, 5:, :], ((0, 0), (0, 0), (0, Ap - A)))

    lane = jnp.arange(Ap, dtype=jnp.int32)
    validA = (lane < A).astype(f32)
    anch = jnp.stack([
        jnp.pad(grid_mask[:, 0], (0, Ap - A)),
        jnp.pad(grid_mask[:, 1], (0, Ap - A)),
        jnp.pad(stride_mask, (0, Ap - A)),
        validA,
    ], axis=0)
    anch = jnp.pad(anch, ((0, 4), (0, 0)))

    tgt = jnp.pad(targets.astype(f32), ((0, 0), (0, 128 - targets.shape[1])))
    toh = jax.nn.one_hot(targets[:, 1].astype(jnp.int32), _C, dtype=f32)

    cost_r, iou_m, cand = pl.pallas_call(
        _phase1_kernel,
        grid=(N, K),
        in_specs=[
            pl.BlockSpec((1, 8, Ac), lambda i, j: (i, 0, j)),
            pl.BlockSpec((1, _C, Ac), lambda i, j: (i, 0, j)),
            pl.BlockSpec((8, Ac), lambda i, j: (0, j)),
            pl.BlockSpec((T, 128), lambda i, j: (0, 0)),
            pl.BlockSpec((T, _C), lambda i, j: (0, 0)),
        ],
        out_specs=[
            pl.BlockSpec((1, T, Ac), lambda i, j: (i, 0, j)),
            pl.BlockSpec((1, T, Ac), lambda i, j: (i, 0, j)),
            pl.BlockSpec((1, 1, T, 128), lambda i, j: (i, j, 0, 0)),
        ],
        out_shape=[
            jax.ShapeDtypeStruct((N, T, Ap), f32),
            jax.ShapeDtypeStruct((N, T, Ap), jnp.int8),
            jax.ShapeDtypeStruct((N, K, T, 128), f32),
        ],
        compiler_params=pltpu.CompilerParams(
            dimension_semantics=("parallel", "parallel")),
    )(pbo, pcls, anch, tgt, toh)

    out = pl.pallas_call(
        _phase2_kernel,
        grid=(N, K),
        in_specs=[
            pl.BlockSpec((1, T, Ac), lambda i, j: (i, 0, j)),
            pl.BlockSpec((1, T, Ac), lambda i, j: (i, 0, j)),
            pl.BlockSpec((1, K, T, 128), lambda i, j: (i, 0, 0, 0)),
            pl.BlockSpec((1, 8, Ac), lambda i, j: (i, 0, j)),
            pl.BlockSpec((1, _C, Ac), lambda i, j: (i, 0, j)),
            pl.BlockSpec((8, Ac), lambda i, j: (0, j)),
            pl.BlockSpec((T, 128), lambda i, j: (0, 0)),
        ],
        out_specs=pl.BlockSpec((1, 1, 1, 128), lambda i, j: (i, j, 0, 0)),
        out_shape=jax.ShapeDtypeStruct((N, K, 1, 128), f32),
        compiler_params=pltpu.CompilerParams(
            dimension_semantics=("parallel", "parallel")),
    )(cost_r, iou_m, cand, pbo, pcls, anch, tgt)

    sums = jnp.sum(out[:, :, 0, :4], axis=(0, 1))
    nfg = sums[0]
    lbox = sums[1] / nfg
    lobj = sums[2] / (nfg * _C)
    lcls = sums[3] / (N * A)
    vec = jnp.stack([lbox, lobj, lcls])
    return lbox + lobj + lcls, vec


# speedup vs baseline: 1.1324x; 1.1324x over previous
import functools

import jax
import jax.numpy as jnp
from jax.experimental import pallas as pl
from jax.experimental.pallas import tpu as pltpu

_C = 80
_EPS_LS = 0.05
_KMAX = 10
_NCHUNK = 2


def _bce_logits(x, t):
    return jnp.maximum(x, 0.0) - x * t + jnp.log1p(jnp.exp(-jnp.abs(x)))


def _atan(z):
    s = jnp.abs(z)
    inv = s > 1.0
    t = jnp.where(inv, 1.0 / jnp.maximum(s, 1e-30), s)
    red = t > 0.41421356
    u = jnp.where(red, (t - 1.0) / (t + 1.0), t)
    u2 = u * u
    p = u * (1.0 + u2 * (-1.0 / 3 + u2 * (1.0 / 5 + u2 * (-1.0 / 7 + u2 * (
        1.0 / 9 + u2 * (-1.0 / 11 + u2 * (1.0 / 13)))))))
    r = jnp.where(red, jnp.pi / 4 + p, p)
    r = jnp.where(inv, jnp.pi / 2 - r, r)
    return jnp.where(z < 0.0, -r, r)


def _lane_slot(vals, slot0, T):
    lane = jax.lax.broadcasted_iota(jnp.int32, (T, 128), 1)
    out = jnp.zeros((T, 128), jnp.float32)
    for q, v in enumerate(vals):
        out = out + jnp.where(lane == slot0 + q, v, 0.0)
    return out


def _phase1_kernel(pbo_ref, pcls_ref, anch_ref, tgt_ref, toh_ref,
                   cr_ref, im_ref, cand_ref):
    f32 = jnp.float32
    T = tgt_ref.shape[0]
    Ac = anch_ref.shape[1]

    img = pl.program_id(0).astype(f32)

    pbo = pbo_ref[0]
    px1 = pbo[0:1, :]
    py1 = pbo[1:2, :]
    px2 = pbo[2:3, :]
    py2 = pbo[3:4, :]
    pobj = pbo[4:5, :]

    gx = anch_ref[0:1, :]
    gy = anch_ref[1:2, :]
    stride = anch_ref[2:3, :]
    xc = (gx + 0.5) * stride
    yc = (gy + 0.5) * stride

    tgt = tgt_ref[...]
    tbi = tgt[:, 0:1]
    tx1 = tgt[:, 2:3]
    ty1 = tgt[:, 3:4]
    tx2 = tgt[:, 4:5]
    ty2 = tgt[:, 5:6]
    tmask = tbi == img

    in_box = (jnp.minimum(jnp.minimum(xc - tx1, yc - ty1),
                          jnp.minimum(tx2 - xc, ty2 - yc))
              > 0.0) & tmask
    cxg = (tx1 + tx2) * 0.5
    cyg = (ty1 + ty2) * 0.5
    in_ctr = (jnp.maximum(jnp.abs(xc - cxg), jnp.abs(yc - cyg))
              < 2.5 * stride) & tmask
    fg = (jnp.sum(in_box.astype(f32), axis=0, keepdims=True) > 0.0) | (
        jnp.sum(in_ctr.astype(f32), axis=0, keepdims=True) > 0.0)
    is_matched = in_box & in_ctr

    area_t = (tx2 - tx1) * (ty2 - ty1)
    area_p = (px2 - px1) * (py2 - py1)
    iw = jnp.clip(jnp.minimum(tx2, px2) - jnp.maximum(tx1, px1), 0.0)
    ih = jnp.clip(jnp.minimum(ty2, py2) - jnp.maximum(ty1, py1), 0.0)
    inter = iw * ih
    iou = inter / (area_t + area_p - inter + 1e-12)

    cls = pcls_ref[0]
    clsp = jax.nn.sigmoid(pobj) * jax.nn.sigmoid(cls)
    cs = jnp.sqrt(jnp.clip(clsp, 1e-12, 1.0))
    lcs = jnp.log(cs + 1e-8)
    l1cs = jnp.log(1.0 - cs + 1e-8)
    s1 = jnp.sum(l1cs, axis=0, keepdims=True)
    d = lcs - l1cs
    sel = jnp.dot(toh_ref[...], d, preferred_element_type=f32)
    bce = -(sel + s1)

    cost = bce - 3.0 * jnp.log(iou + 1e-8) \
        + 100000.0 * (1.0 - is_matched.astype(f32))

    iou_m = jnp.where(fg, iou, 0.0)
    cost_r = jnp.where(fg, cost, 1e9)
    cr_ref[0] = cost_r
    im_ref[0] = (iou > 0.0).astype(jnp.int8)

    idx = jax.lax.broadcasted_iota(jnp.int32, (T, Ac), 1).astype(f32)

    x = iou_m
    ivals, icnts = [], []
    for _ in range(_KMAX):
        m = jnp.max(x, axis=1, keepdims=True)
        eq = x == m
        cnt = jnp.sum(eq.astype(f32), axis=1, keepdims=True)
        ivals.append(m)
        icnts.append(cnt)
        x = jnp.where(eq, -1e30, x)

    y = cost_r
    cvals, cidx = [], []
    for _ in range(_KMAX):
        m = jnp.min(y, axis=1, keepdims=True)
        eq = y == m
        imn = jnp.min(jnp.where(eq, idx, float(Ac)), axis=1, keepdims=True)
        cvals.append(m)
        cidx.append(imn)
        y = jnp.where(idx == imn, 1e30, y)

    amc = jnp.sum(fg.astype(f32))
    cand = (_lane_slot(ivals, 0, T)
            + _lane_slot(cvals, 16, T)
            + _lane_slot(cidx, 32, T)
            + _lane_slot([jnp.full((T, 1), amc, f32)], 48, T)
            + _lane_slot(icnts, 64, T))
    cand_ref[0, 0] = cand


def _phase2_kernel(cr_ref, im_ref, cand_ref, pbo_ref, pcls_ref, anch_ref,
                   tgt_ref, out_ref):
    f32 = jnp.float32
    T = tgt_ref.shape[0]
    Ac = anch_ref.shape[1]
    K = cand_ref.shape[1]

    img = pl.program_id(0).astype(f32)
    chunk = pl.program_id(1)

    tgt = tgt_ref[...]
    tmask = tgt[:, 0:1] == img

    cost_r = cr_ref[0]
    ioupos = im_ref[0] != 0

    iou_c = jnp.concatenate([cand_ref[0, k][:, 0:_KMAX] for k in range(K)],
                            axis=1)
    icnt_c = jnp.concatenate([cand_ref[0, k][:, 64:64 + _KMAX]
                              for k in range(K)], axis=1)
    cv_c = jnp.concatenate([cand_ref[0, k][:, 16:16 + _KMAX]
                            for k in range(K)], axis=1)
    base = jnp.concatenate(
        [jnp.full((T, _KMAX), float(k * Ac), f32) for k in range(K)], axis=1)
    ci_c = jnp.concatenate([cand_ref[0, k][:, 32:32 + _KMAX]
                            for k in range(K)], axis=1) + base
    am = functools.reduce(
        lambda a, b: a + b,
        [cand_ref[0, k][0:1, 48:49] for k in range(K)])

    x = iou_c
    acc = jnp.zeros((T, 1), f32)
    krem = jnp.full((T, 1), float(_KMAX), f32)
    for _ in range(_KMAX):
        m = jnp.max(x, axis=1, keepdims=True)
        eq = x == m
        csum = jnp.sum(jnp.where(eq, icnt_c, 0.0), axis=1, keepdims=True)
        take = jnp.minimum(csum, krem)
        acc = acc + m * take
        krem = krem - take
        x = jnp.where(eq, -1e30, x)
    dks = jnp.clip(jnp.floor(acc), 1.0, am)

    y = cv_c
    tv = jnp.full((T, 1), -1e30, f32)
    ti = jnp.full((T, 1), -1.0, f32)
    for k in range(_KMAX):
        m = jnp.min(y, axis=1, keepdims=True)
        eq = y == m
        iv = jnp.min(jnp.where(eq, ci_c, 1e9), axis=1, keepdims=True)
        hit = dks == float(k + 1)
        tv = jnp.where(hit, m, tv)
        ti = jnp.where(hit, iv, ti)
        y = jnp.where(ci_c == iv, 1e30, y)

    idxg = (jax.lax.broadcasted_iota(jnp.int32, (T, Ac), 1)
            + chunk * Ac).astype(f32)
    matching = ((cost_r < tv) | ((cost_r == tv) & (idxg <= ti))) & tmask

    mf = matching.astype(f32)
    agt = jnp.sum(mf, axis=0, keepdims=True)
    conflict = agt > 1.0
    cost_b = jnp.where(tmask, cost_r, 1e12)
    mb = jnp.min(cost_b, axis=0, keepdims=True)
    tio = jax.lax.broadcasted_iota(jnp.int32, (T, Ac), 0).astype(f32)
    best = jnp.min(jnp.where(cost_b == mb, tio, float(T)),
                   axis=0, keepdims=True)
    onehot_best = (tio == best).astype(f32)
    mf = jnp.where(conflict, onehot_best, mf)

    fgf = (jnp.sum(mf, axis=0, keepdims=True) > 0.0).astype(f32)
    piou_pos = jnp.sum(jnp.where(ioupos, mf, 0.0), axis=0,
                       keepdims=True) > 0.0

    tattr = jax.lax.dot_general(tgt, mf, (((0,), (0,)), ((), ())),
                                preferred_element_type=f32)
    cidm = tattr[1:2, :]
    rx1 = tattr[2:3, :]
    ry1 = tattr[3:4, :]
    rx2 = tattr[4:5, :]
    ry2 = tattr[5:6, :]

    pbo = pbo_ref[0]
    px1 = pbo[0:1, :]
    py1 = pbo[1:2, :]
    px2 = pbo[2:3, :]
    py2 = pbo[3:4, :]
    pobj = pbo[4:5, :]
    valid = anch_ref[3:4, :]

    eps = 1e-7
    pa = (px2 - px1) * (py2 - py1)
    ta = (rx2 - rx1) * (ry2 - ry1)
    ciw = jnp.clip(jnp.minimum(px2, rx2) - jnp.maximum(px1, rx1), 0.0)
    cih = jnp.clip(jnp.minimum(py2, ry2) - jnp.maximum(py1, ry1), 0.0)
    cinter = ciw * cih
    cunion = pa + ta - cinter + eps
    ciou_i = cinter / cunion
    cw = jnp.maximum(px2, rx2) - jnp.minimum(px1, rx1)
    ch = jnp.maximum(py2, ry2) - jnp.minimum(py1, ry1)
    c2 = cw * cw + ch * ch + eps
    rho2 = ((px1 + px2 - rx1 - rx2) ** 2 + (py1 + py2 - ry1 - ry2) ** 2) / 4.0
    v = (4.0 / (jnp.pi ** 2)) * (
        _atan((rx2 - rx1) / (ry2 - ry1 + eps))
        - _atan((px2 - px1) / (py2 - py1 + eps))) ** 2
    alpha = v / (1.0 - ciou_i + v + eps)
    ciou_full = ciou_i - rho2 / c2 - alpha * v
    lbox_i = jnp.sum((1.0 - ciou_full) * fgf)

    cls = pcls_ref[0]
    crow = jax.lax.broadcasted_iota(jnp.int32, (_C, Ac), 0).astype(f32)
    cls_t = jnp.where((crow == cidm) & piou_pos,
                      1.0 - _EPS_LS, _EPS_LS / (_C - 1))
    lcls_i = jnp.sum(jnp.sum(_bce_logits(cls, cls_t), axis=0,
                             keepdims=True) * fgf)
    lobj_i = jnp.sum(_bce_logits(pobj, fgf) * valid)
    nfg_i = jnp.sum(fgf)

    lane = jax.lax.broadcasted_iota(jnp.int32, (1, 128), 1)
    out_ref[0, 0] = (jnp.where(lane == 0, nfg_i, 0.0)
                     + jnp.where(lane == 1, lbox_i, 0.0)
                     + jnp.where(lane == 2, lcls_i, 0.0)
                     + jnp.where(lane == 3, lobj_i, 0.0))


@jax.jit
def kernel(preds, grid_mask, stride_mask, targets):
    f32 = jnp.float32
    N, A, _ = preds.shape
    T = targets.shape[0]
    K = _NCHUNK
    Ap = ((A + 128 * K - 1) // (128 * K)) * (128 * K)
    Ac = Ap // K

    pt = jnp.transpose(preds, (0, 2, 1))
    pbo = jnp.pad(pt[:, :5, :], ((0, 0), (0, 3), (0, Ap - A)))
    pcls = jnp.pad(pt[:, 5:, :], ((0, 0), (0, 0), (0, Ap - A)))

    lane = jnp.arange(Ap, dtype=jnp.int32)
    validA = (lane < A).astype(f32)
    anch = jnp.stack([
        jnp.pad(grid_mask[:, 0], (0, Ap - A)),
        jnp.pad(grid_mask[:, 1], (0, Ap - A)),
        jnp.pad(stride_mask, (0, Ap - A)),
        validA,
    ], axis=0)
    anch = jnp.pad(anch, ((0, 4), (0, 0)))

    tgt = jnp.pad(targets.astype(f32), ((0, 0), (0, 128 - targets.shape[1])))
    toh = jax.nn.one_hot(targets[:, 1].astype(jnp.int32), _C, dtype=f32)

    cost_r, iou_m, cand = pl.pallas_call(
        _phase1_kernel,
        grid=(N, K),
        in_specs=[
            pl.BlockSpec((1, 8, Ac), lambda i, j: (i, 0, j)),
            pl.BlockSpec((1, _C, Ac), lambda i, j: (i, 0, j)),
            pl.BlockSpec((8, Ac), lambda i, j: (0, j)),
            pl.BlockSpec((T, 128), lambda i, j: (0, 0)),
            pl.BlockSpec((T, _C), lambda i, j: (0, 0)),
        ],
        out_specs=[
            pl.BlockSpec((1, T, Ac), lambda i, j: (i, 0, j)),
            pl.BlockSpec((1, T, Ac), lambda i, j: (i, 0, j)),
            pl.BlockSpec((1, 1, T, 128), lambda i, j: (i, j, 0, 0)),
        ],
        out_shape=[
            jax.ShapeDtypeStruct((N, T, Ap), f32),
            jax.ShapeDtypeStruct((N, T, Ap), jnp.int8),
            jax.ShapeDtypeStruct((N, K, T, 128), f32),
        ],
        compiler_params=pltpu.CompilerParams(
            dimension_semantics=("parallel", "parallel")),
    )(pbo, pcls, anch, tgt, toh)

    out = pl.pallas_call(
        _phase2_kernel,
        grid=(N, K),
        in_specs=[
            pl.BlockSpec((1, T, Ac), lambda i, j: (i, 0, j)),
            pl.BlockSpec((1, T, Ac), lambda i, j: (i, 0, j)),
            pl.BlockSpec((1, K, T, 128), lambda i, j: (i, 0, 0, 0)),
            pl.BlockSpec((1, 8, Ac), lambda i, j: (i, 0, j)),
            pl.BlockSpec((1, _C, Ac), lambda i, j: (i, 0, j)),
            pl.BlockSpec((8, Ac), lambda i, j: (0, j)),
            pl.BlockSpec((T, 128), lambda i, j: (0, 0)),
        ],
        out_specs=pl.BlockSpec((1, 1, 1, 128), lambda i, j: (i, j, 0, 0)),
        out_shape=jax.ShapeDtypeStruct((N, K, 1, 128), f32),
        compiler_params=pltpu.CompilerParams(
            dimension_semantics=("parallel", "parallel")),
    )(cost_r, iou_m, cand, pbo, pcls, anch, tgt)

    sums = jnp.sum(out[:, :, 0, :4], axis=(0, 1))
    nfg = sums[0]
    lbox = sums[1] / nfg
    lobj = sums[2] / (nfg * _C)
    lcls = sums[3] / (N * A)
    vec = jnp.stack([lbox, lobj, lcls])
    return lbox + lobj + lcls, vec
